# baseline (speedup 1.0000x reference)
import jax
import jax.numpy as jnp
from jax import lax
from jax.experimental import pallas as pl
from jax.experimental.pallas import tpu as pltpu
from jax.experimental.pallas import tpu_sc as plsc

NUM_PATCHES = 1025
HIDDEN_DIM = 1280
MAX_NUM_TILES = 4
NUM_TABLE_ROWS = 9
SLAB = NUM_PATCHES * HIDDEN_DIM
NCHUNK = 250
CHUNK = SLAB // NCHUNK
CROWS = CHUNK // 128
NS = 16


def _sc_body(x_hbm, emb_hbm, tt_hbm, idx_hbm, coef_hbm, out_hbm,
             xb0, tb0, eb0, ob0, xb1, tb1, eb1, ob1,
             idx_v, cbuf, si0, si1, so0, so1, sg0, sg1):
    b = lax.axis_index("c") * NS + lax.axis_index("s")
    pltpu.sync_copy(idx_hbm.at[b], idx_v)
    pltpu.sync_copy(coef_hbm, cbuf)
    c0 = cbuf[0]
    c1 = cbuf[1]
    sets = ((xb0, tb0, eb0, ob0, si0, so0, sg0), (xb1, tb1, eb1, ob1, si1, so1, sg1))

    def start_in(c, xb, tb, eb, si, sg):
        pltpu.async_copy(x_hbm.at[b, c], xb, si)
        pltpu.async_copy(tt_hbm.at[idx_v.at[c]], tb, sg)
        pltpu.async_copy(emb_hbm.at[c], eb, si)

    def wait_in(c, xb, tb, eb, si, sg):
        pltpu.make_async_copy(x_hbm.at[b, c], xb, si).wait()
        pltpu.make_async_copy(tt_hbm.at[idx_v.at[c]], tb, sg).wait()
        pltpu.make_async_copy(emb_hbm.at[c], eb, si).wait()

    start_in(0, xb0, tb0, eb0, si0, sg0)
    start_in(1, xb1, tb1, eb1, si1, sg1)

    def outer(g, carry):
        for s in (0, 1):
            xb, tb, eb, ob, si, so, sg = sets[s]
            c = 2 * g + s
            wait_in(c, xb, tb, eb, si, sg)

            @pl.when(c >= 2)
            def _():
                pltpu.make_async_copy(ob, out_hbm.at[b, c], so).wait()

            def row(i, rcarry):
                for j in range(8):
                    sl = pl.ds(16 * j, 16)
                    ob[i, sl] = xb[i, sl] + c0 * eb[i, sl] + c1 * tb[i, sl]
                return rcarry

            lax.fori_loop(0, CROWS, row, 0)
            pltpu.async_copy(ob, out_hbm.at[b, c], so)

            @pl.when(c + 2 < NCHUNK)
            def _():
                start_in(c + 2, xb, tb, eb, si, sg)

        return carry

    lax.fori_loop(0, NCHUNK // 2, outer, 0)
    pltpu.make_async_copy(ob0, out_hbm.at[b, NCHUNK - 2], so0).wait()
    pltpu.make_async_copy(ob1, out_hbm.at[b, NCHUNK - 1], so1).wait()


def kernel(x, aspect_ratio_ids, tile_indices, embedding, gate, tile_table):
    bt = x.shape[0]
    t = jnp.tanh(gate)[0]
    coef = jnp.stack([jnp.full((16,), 1.0, jnp.float32) - t,
                      jnp.full((16,), 0.0, jnp.float32) + t])
    slab_rows = SLAB // 128
    base = (aspect_ratio_ids.astype(jnp.int32) * MAX_NUM_TILES
            + tile_indices.astype(jnp.int32)) * slab_rows
    idx_mat = (base[:, None, None]
               + jnp.arange(slab_rows, dtype=jnp.int32).reshape(NCHUNK, CROWS)[None])

    x4 = x.reshape(bt, NCHUNK, CROWS, 128)
    emb4 = embedding.reshape(NCHUNK, CROWS, 128)
    tt2 = tile_table.reshape(NUM_TABLE_ROWS * MAX_NUM_TILES * slab_rows, 128)

    mesh = plsc.VectorSubcoreMesh(core_axis_name="c", subcore_axis_name="s")
    buf = lambda: pltpu.VMEM((CROWS, 128), jnp.float32)
    out = pl.kernel(
        _sc_body,
        out_type=jax.ShapeDtypeStruct((bt, NCHUNK, CROWS, 128), jnp.float32),
        mesh=mesh,
        scratch_types=[
            buf(), buf(), buf(), buf(),
            buf(), buf(), buf(), buf(),
            pltpu.VMEM((NCHUNK, CROWS), jnp.int32),
            pltpu.VMEM((2, 16), jnp.float32),
            pltpu.SemaphoreType.DMA,
            pltpu.SemaphoreType.DMA,
            pltpu.SemaphoreType.DMA,
            pltpu.SemaphoreType.DMA,
            pltpu.SemaphoreType.DMA,
            pltpu.SemaphoreType.DMA,
        ],
    )(x4, emb4, tt2, idx_mat, coef)
    return out.reshape(bt, NUM_PATCHES, HIDDEN_DIM)


# speedup vs baseline: 1.3737x; 1.3737x over previous
import jax
import jax.numpy as jnp
from jax import lax
from jax.experimental import pallas as pl
from jax.experimental.pallas import tpu as pltpu
from jax.experimental.pallas import tpu_sc as plsc

NUM_PATCHES = 1025
HIDDEN_DIM = 1280
MAX_NUM_TILES = 4
NUM_TABLE_ROWS = 9
SLAB = NUM_PATCHES * HIDDEN_DIM
NCHUNK = 250
CHUNK = SLAB // NCHUNK
CROWS = CHUNK // 128
NS = 16


def _sc_body(x_hbm, emb_hbm, tt_hbm, idx_hbm, coef_hbm, out_hbm,
             xb0, tb0, eb0, ob0, xb1, tb1, eb1, ob1,
             idx_v, cbuf, si0, si1, so0, so1, sg0, sg1):
    b = lax.axis_index("c") * NS + lax.axis_index("s")
    pltpu.sync_copy(idx_hbm, idx_v)
    pltpu.sync_copy(coef_hbm, cbuf)
    c0 = cbuf[0]
    c1 = cbuf[1]
    row0 = idx_v[b][0]
    sets = ((xb0, tb0, eb0, ob0, si0, so0, sg0), (xb1, tb1, eb1, ob1, si1, so1, sg1))

    def start_in(c, xb, tb, eb, si, sg):
        pltpu.async_copy(x_hbm.at[b, c], xb, si)
        pltpu.async_copy(tt_hbm.at[pl.ds((row0 + c * CROWS) * 128, CHUNK)], tb, sg)
        pltpu.async_copy(emb_hbm.at[c], eb, si)

    def wait_in(c, xb, tb, eb, si, sg):
        pltpu.make_async_copy(x_hbm.at[b, c], xb, si).wait()
        pltpu.make_async_copy(tt_hbm.at[pl.ds((row0 + c * CROWS) * 128, CHUNK)], tb, sg).wait()
        pltpu.make_async_copy(emb_hbm.at[c], eb, si).wait()

    start_in(0, xb0, tb0, eb0, si0, sg0)
    start_in(1, xb1, tb1, eb1, si1, sg1)

    def outer(g, carry):
        for s in (0, 1):
            xb, tb, eb, ob, si, so, sg = sets[s]
            c = 2 * g + s
            wait_in(c, xb, tb, eb, si, sg)

            @pl.when(c >= 2)
            def _():
                pltpu.make_async_copy(ob, out_hbm.at[b, c], so).wait()

            def row(i, rcarry):
                for j in range(8):
                    sl = pl.ds(16 * j, 16)
                    tv = tb[pl.ds(i * 128 + 16 * j, 16)]
                    ob[i, sl] = xb[i, sl] + c0 * eb[i, sl] + c1 * tv
                return rcarry

            lax.fori_loop(0, CROWS, row, 0)
            pltpu.async_copy(ob, out_hbm.at[b, c], so)

            @pl.when(c + 2 < NCHUNK)
            def _():
                start_in(c + 2, xb, tb, eb, si, sg)

        return carry

    lax.fori_loop(0, NCHUNK // 2, outer, 0)
    pltpu.make_async_copy(ob0, out_hbm.at[b, NCHUNK - 2], so0).wait()
    pltpu.make_async_copy(ob1, out_hbm.at[b, NCHUNK - 1], so1).wait()


def kernel(x, aspect_ratio_ids, tile_indices, embedding, gate, tile_table):
    bt = x.shape[0]
    t = jnp.tanh(gate)[0]
    coef = jnp.stack([jnp.full((16,), 1.0, jnp.float32) - t,
                      jnp.full((16,), 0.0, jnp.float32) + t])
    slab_rows = SLAB // 128
    base = (aspect_ratio_ids.astype(jnp.int32) * MAX_NUM_TILES
            + tile_indices.astype(jnp.int32)) * slab_rows
    idx_mat = jnp.broadcast_to(base[:, None], (bt, NS))

    x4 = x.reshape(bt, NCHUNK, CROWS, 128)
    emb4 = embedding.reshape(NCHUNK, CROWS, 128)
    tt1 = tile_table.reshape(NUM_TABLE_ROWS * MAX_NUM_TILES * SLAB)

    mesh = plsc.VectorSubcoreMesh(core_axis_name="c", subcore_axis_name="s")
    buf = lambda: pltpu.VMEM((CROWS, 128), jnp.float32)
    tbuf = lambda: pltpu.VMEM((CHUNK,), jnp.float32)
    out = pl.kernel(
        _sc_body,
        out_type=jax.ShapeDtypeStruct((bt, NCHUNK, CROWS, 128), jnp.float32),
        mesh=mesh,
        scratch_types=[
            buf(), tbuf(), buf(), buf(),
            buf(), tbuf(), buf(), buf(),
            pltpu.VMEM((32, NS), jnp.int32),
            pltpu.VMEM((2, 16), jnp.float32),
            pltpu.SemaphoreType.DMA,
            pltpu.SemaphoreType.DMA,
            pltpu.SemaphoreType.DMA,
            pltpu.SemaphoreType.DMA,
            pltpu.SemaphoreType.DMA,
            pltpu.SemaphoreType.DMA,
        ],
    )(x4, emb4, tt1, idx_mat, coef)
    return out.reshape(bt, NUM_PATCHES, HIDDEN_DIM)


# speedup vs baseline: 1.4020x; 1.0206x over previous
import jax
import jax.numpy as jnp
from jax import lax
from jax.experimental import pallas as pl
from jax.experimental.pallas import tpu as pltpu
from jax.experimental.pallas import tpu_sc as plsc

NUM_PATCHES = 1025
HIDDEN_DIM = 1280
MAX_NUM_TILES = 4
NUM_TABLE_ROWS = 9
SLAB = NUM_PATCHES * HIDDEN_DIM
PCH = 8
NCHUNK = NUM_PATCHES // PCH
CHUNK = PCH * HIDDEN_DIM
NLG = HIDDEN_DIM // 16
NS = 16


def _sc_body(x_hbm, emb_hbm, tt_hbm, idx_hbm, coef_hbm, out_hbm,
             xb0, tb0, eb0, ob0, xb1, tb1, eb1, ob1,
             idx_v, cbuf, si0, si1, so0, so1, sg0, sg1):
    b = lax.axis_index("c") * NS + lax.axis_index("s")
    pltpu.sync_copy(idx_hbm, idx_v)
    pltpu.sync_copy(coef_hbm, cbuf)
    c0 = cbuf[0]
    c1 = cbuf[1]
    base = pl.multiple_of(idx_v[b][0], 128)
    sets = ((xb0, tb0, eb0, ob0, si0, so0, sg0), (xb1, tb1, eb1, ob1, si1, so1, sg1))

    def start_in(c, xb, tb, eb, si, sg):
        pltpu.async_copy(x_hbm.at[b, pl.ds(PCH * c, PCH)], xb, si)
        pltpu.async_copy(tt_hbm.at[pl.ds(base + c * CHUNK, CHUNK)], tb, sg)
        pltpu.async_copy(emb_hbm.at[pl.ds(PCH * c, PCH)], eb, si)

    def wait_in(c, xb, tb, eb, si, sg):
        pltpu.make_async_copy(x_hbm.at[b, pl.ds(PCH * c, PCH)], xb, si).wait()
        pltpu.make_async_copy(tt_hbm.at[pl.ds(base + c * CHUNK, CHUNK)], tb, sg).wait()
        pltpu.make_async_copy(emb_hbm.at[pl.ds(PCH * c, PCH)], eb, si).wait()

    start_in(0, xb0, tb0, eb0, si0, sg0)
    start_in(1, xb1, tb1, eb1, si1, sg1)

    def outer(g, carry):
        for s in (0, 1):
            xb, tb, eb, ob, si, so, sg = sets[s]
            c = 2 * g + s
            wait_in(c, xb, tb, eb, si, sg)

            @pl.when(c >= 2)
            def _():
                pltpu.make_async_copy(ob, out_hbm.at[b, pl.ds(0, PCH)], so).wait()

            def lanegrp(j, rcarry):
                sl = pl.ds(16 * j, 16)
                for i in range(PCH):
                    tv = tb[pl.ds(i * HIDDEN_DIM + 16 * j, 16)]
                    ob[i, sl] = xb[i, sl] + c0 * eb[i, sl] + c1 * tv
                return rcarry

            lax.fori_loop(0, NLG, lanegrp, 0)
            pltpu.async_copy(ob, out_hbm.at[b, pl.ds(PCH * c, PCH)], so)

            @pl.when(c + 2 < NCHUNK)
            def _():
                start_in(c + 2, xb, tb, eb, si, sg)

        return carry

    lax.fori_loop(0, NCHUNK // 2, outer, 0)
    pltpu.make_async_copy(ob0, out_hbm.at[b, pl.ds(0, PCH)], so0).wait()
    pltpu.make_async_copy(ob1, out_hbm.at[b, pl.ds(0, PCH)], so1).wait()

    last = NUM_PATCHES - 1
    pltpu.sync_copy(x_hbm.at[b, pl.ds(last, 1)], xb0.at[pl.ds(0, 1)])
    pltpu.sync_copy(tt_hbm.at[pl.ds(base + last * HIDDEN_DIM, HIDDEN_DIM)],
                    tb0.at[pl.ds(0, HIDDEN_DIM)])
    pltpu.sync_copy(emb_hbm.at[pl.ds(last, 1)], eb0.at[pl.ds(0, 1)])

    def tail_lanegrp(j, rcarry):
        sl = pl.ds(16 * j, 16)
        ob0[0, sl] = xb0[0, sl] + c0 * eb0[0, sl] + c1 * tb0[pl.ds(16 * j, 16)]
        return rcarry

    lax.fori_loop(0, NLG, tail_lanegrp, 0)
    pltpu.sync_copy(ob0.at[pl.ds(0, 1)], out_hbm.at[b, pl.ds(last, 1)])


def kernel(x, aspect_ratio_ids, tile_indices, embedding, gate, tile_table):
    bt = x.shape[0]
    t = jnp.tanh(gate)[0]
    coef = jnp.stack([jnp.full((16,), 1.0, jnp.float32) - t,
                      jnp.full((16,), 0.0, jnp.float32) + t])
    row_floats = tile_table.shape[1]
    base = (aspect_ratio_ids.astype(jnp.int32) * row_floats
            + tile_indices.astype(jnp.int32) * SLAB)
    idx_mat = jnp.broadcast_to(base[:, None], (bt, NS))
    tt1 = tile_table.reshape(NUM_TABLE_ROWS * MAX_NUM_TILES * SLAB)

    mesh = plsc.VectorSubcoreMesh(core_axis_name="c", subcore_axis_name="s")
    buf = lambda: pltpu.VMEM((PCH, HIDDEN_DIM), jnp.float32)
    tbuf = lambda: pltpu.VMEM((CHUNK,), jnp.float32)
    out = pl.kernel(
        _sc_body,
        out_type=jax.ShapeDtypeStruct((bt, NUM_PATCHES, HIDDEN_DIM), jnp.float32),
        mesh=mesh,
        scratch_types=[
            buf(), tbuf(), buf(), buf(),
            buf(), tbuf(), buf(), buf(),
            pltpu.VMEM((bt, NS), jnp.int32),
            pltpu.VMEM((2, 16), jnp.float32),
            pltpu.SemaphoreType.DMA,
            pltpu.SemaphoreType.DMA,
            pltpu.SemaphoreType.DMA,
            pltpu.SemaphoreType.DMA,
            pltpu.SemaphoreType.DMA,
            pltpu.SemaphoreType.DMA,
        ],
    )(x, embedding, tt1, idx_mat, coef)
    return out
